# baseline (speedup 1.0000x reference)
import functools

import jax
import jax.numpy as jnp
from jax import lax
from jax.experimental import pallas as pl
from jax.experimental.pallas import tpu as pltpu
from jax.experimental.pallas import tpu_sc as plsc

_E = 8
_TOPK = 2
_TM = 256
_LANES = 128


def _router_body(x_ref, wg_ref, idx_ref, sc_ref):
    x = x_ref[...]
    wg = wg_ref[...]
    logits = jnp.dot(x, wg, preferred_element_type=jnp.float32)
    col = lax.broadcasted_iota(jnp.int32, logits.shape, 1)
    valid = col < _E
    logits = jnp.where(valid, logits, jnp.float32(-1e30))
    m = jnp.max(logits, axis=1, keepdims=True)
    p = jnp.exp(logits - m)
    p = jnp.where(valid, p, 0.0)
    p = p / jnp.sum(p, axis=1, keepdims=True)
    s1 = jnp.max(p, axis=1, keepdims=True)
    i1 = jnp.min(jnp.where(p == s1, col, _LANES), axis=1, keepdims=True)
    p2 = jnp.where(col == i1, jnp.float32(-1.0), p)
    s2 = jnp.max(p2, axis=1, keepdims=True)
    i2 = jnp.min(jnp.where(p2 == s2, col, _LANES), axis=1, keepdims=True)
    idx_ref[...] = jnp.where(col == 0, i1, jnp.where(col == 1, i2, 0))
    sc_ref[...] = jnp.where(col == 0, s1, jnp.where(col == 1, s2, 0.0))


def _run_router(x_flat, W_g):
    n, d = x_flat.shape
    wg_pad = jnp.zeros((d, _LANES), jnp.float32).at[:, :_E].set(W_g)
    grid = (n // _TM,)
    return pl.pallas_call(
        _router_body,
        grid=grid,
        in_specs=[
            pl.BlockSpec((_TM, d), lambda i: (i, 0)),
            pl.BlockSpec((d, _LANES), lambda i: (0, 0)),
        ],
        out_specs=[
            pl.BlockSpec((_TM, _LANES), lambda i: (i, 0)),
            pl.BlockSpec((_TM, _LANES), lambda i: (i, 0)),
        ],
        out_shape=[
            jax.ShapeDtypeStruct((n, _LANES), jnp.int32),
            jax.ShapeDtypeStruct((n, _LANES), jnp.float32),
        ],
    )(x_flat, wg_pad)


def _sc_gather(table, idx):
    n_rows = idx.shape[0]
    d = table.shape[1]
    info = plsc.get_sparse_core_info()
    nw = info.num_cores * info.num_subcores
    per_w = n_rows // nw
    ch = next(c for c in (24, 16, 8) if per_w % c == 0)
    n_ch = per_w // ch
    mesh = plsc.VectorSubcoreMesh(core_axis_name="c", subcore_axis_name="s")

    @functools.partial(
        pl.kernel,
        mesh=mesh,
        out_type=jax.ShapeDtypeStruct((n_rows, d), jnp.float32),
        scratch_types=[
            pltpu.VMEM((per_w,), jnp.int32),
            pltpu.VMEM((2, ch, d), jnp.float32),
            pltpu.SemaphoreType.DMA,
            pltpu.SemaphoreType.DMA,
            pltpu.SemaphoreType.DMA,
        ],
    )
    def k(table_hbm, idx_hbm, out_hbm, idx_v, rows_v, gsem0, gsem1, osem):
        wid = lax.axis_index("s") * info.num_cores + lax.axis_index("c")
        base = wid * per_w
        pltpu.sync_copy(idx_hbm.at[pl.ds(base, per_w)], idx_v)
        gsems = [gsem0, gsem1]

        def gath(c):
            return pltpu.async_copy(
                table_hbm.at[idx_v.at[pl.ds(c * ch, ch)]],
                rows_v.at[c % 2],
                gsems[c % 2],
            )

        gath(0)
        outs = []
        for c in range(n_ch):
            pltpu.make_async_copy(
                table_hbm.at[idx_v.at[pl.ds(c * ch, ch)]],
                rows_v.at[c % 2],
                gsems[c % 2],
            ).wait()
            if c + 1 < n_ch:
                if c - 1 >= 0:
                    outs[c - 1].wait()
                gath(c + 1)
            outs.append(
                pltpu.async_copy(
                    rows_v.at[c % 2],
                    out_hbm.at[pl.ds(base + c * ch, ch)],
                    osem,
                )
            )
        outs[n_ch - 2].wait()
        outs[n_ch - 1].wait()

    return k(table, idx)


def _ffn_body(se_ref, sb_ref, sbo_ref, x_ref, wg_ref, wu_ref, wd_ref, out_ref):
    del sb_ref, sbo_ref
    s = pl.program_id(0)

    @pl.when(se_ref[s] > 0)
    def _work():
        x = x_ref[...]
        g = jnp.dot(x, wg_ref[0], preferred_element_type=jnp.float32)
        u = jnp.dot(x, wu_ref[0], preferred_element_type=jnp.float32)
        h = (g * jax.nn.sigmoid(g)) * u
        out_ref[...] = jnp.dot(h, wd_ref[0], preferred_element_type=jnp.float32)


def _run_ffn(x_sorted, W_gate, W_up, W_down, step_e, step_b, step_bo, max_steps):
    s_pad, d = x_sorted.shape
    f = W_gate.shape[-1]
    grid_spec = pltpu.PrefetchScalarGridSpec(
        num_scalar_prefetch=3,
        grid=(max_steps,),
        in_specs=[
            pl.BlockSpec((_TM, d), lambda s, se, sb, sbo: (sb[s], 0)),
            pl.BlockSpec((1, d, f), lambda s, se, sb, sbo: (jnp.abs(se[s]) - 1, 0, 0)),
            pl.BlockSpec((1, d, f), lambda s, se, sb, sbo: (jnp.abs(se[s]) - 1, 0, 0)),
            pl.BlockSpec((1, f, d), lambda s, se, sb, sbo: (jnp.abs(se[s]) - 1, 0, 0)),
        ],
        out_specs=pl.BlockSpec((_TM, d), lambda s, se, sb, sbo: (sbo[s], 0)),
    )
    return pl.pallas_call(
        _ffn_body,
        grid_spec=grid_spec,
        out_shape=jax.ShapeDtypeStruct((s_pad + _TM, d), jnp.float32),
    )(step_e, step_b, step_bo, x_sorted, W_gate, W_up, W_down)


def _comb_body(x_ref, wsg_ref, wsu_ref, wsd_ref, a_ref, b_ref, sc_ref, out_ref):
    x = x_ref[...]
    g = jnp.dot(x, wsg_ref[...], preferred_element_type=jnp.float32)
    u = jnp.dot(x, wsu_ref[...], preferred_element_type=jnp.float32)
    h = (g * jax.nn.sigmoid(g)) * u
    out = jnp.dot(h, wsd_ref[...], preferred_element_type=jnp.float32)
    out_ref[...] = out + a_ref[...] * sc_ref[:, 0:1] + b_ref[...] * sc_ref[:, 1:2]


def _run_combine(x_flat, Wsg, Wsu, Wsd, ab_rows, sc_out):
    n, d = x_flat.shape
    f = Wsg.shape[-1]
    n_t = n // _TM
    return pl.pallas_call(
        _comb_body,
        grid=(n_t,),
        in_specs=[
            pl.BlockSpec((_TM, d), lambda i: (i, 0)),
            pl.BlockSpec((d, f), lambda i: (0, 0)),
            pl.BlockSpec((d, f), lambda i: (0, 0)),
            pl.BlockSpec((f, d), lambda i: (0, 0)),
            pl.BlockSpec((_TM, d), lambda i: (i, 0)),
            pl.BlockSpec((_TM, d), lambda i, n_t=n_t: (i + n_t, 0)),
            pl.BlockSpec((_TM, _LANES), lambda i: (i, 0)),
        ],
        out_specs=pl.BlockSpec((_TM, d), lambda i: (i, 0)),
        out_shape=jax.ShapeDtypeStruct((n, d), jnp.float32),
    )(x_flat, Wsg, Wsu, Wsd, ab_rows, ab_rows, sc_out)


def kernel(x, W_g, W_gate, W_up, W_down, W_shared_gate, W_shared_up, W_shared_down):
    bx, tx, d = x.shape
    n = bx * tx
    x_flat = x.reshape(n, d)
    s = n * _TOPK
    s_pad = s + _E * _TM
    max_steps = s_pad // _TM
    maxt = n // _TM

    idx_out, sc_out = _run_router(x_flat, W_g)

    flat_e = idx_out[:, :_TOPK].reshape(-1)
    flat_t = (jnp.arange(s, dtype=jnp.int32) // _TOPK).astype(jnp.int32)
    order = jnp.argsort(flat_e, stable=True)
    sorted_e = flat_e[order]
    sizes = jnp.bincount(flat_e, length=_E)
    start = jnp.concatenate([jnp.zeros((1,), sizes.dtype), jnp.cumsum(sizes)[:-1]])
    al_sizes = ((sizes + _TM - 1) // _TM) * _TM
    al_start = jnp.concatenate(
        [jnp.zeros((1,), al_sizes.dtype), jnp.cumsum(al_sizes)[:-1]]
    )
    rank = jnp.arange(s, dtype=sizes.dtype) - start[sorted_e]
    dest = (al_start[sorted_e] + rank).astype(jnp.int32)
    inv = jnp.zeros((s,), jnp.int32).at[order].set(dest)
    ab_idx = jnp.concatenate([inv[0::2], inv[1::2]])
    p_arange = jnp.arange(s_pad, dtype=jnp.int32)
    e_p = jnp.sum(
        (p_arange[:, None] >= al_start[None, :]).astype(jnp.int32), axis=1
    ) - 1
    off_p = p_arange - al_start[e_p]
    valid = off_p < sizes[e_p]
    j_p = jnp.where(valid, off_p + start[e_p], 0)
    pad_idx = (p_arange * 7) % n
    gidx = jnp.where(valid, flat_t[order[j_p]], pad_idx).astype(jnp.int32)

    e_grid = jnp.repeat(jnp.arange(_E, dtype=jnp.int32), maxt)
    t_grid = jnp.tile(jnp.arange(maxt, dtype=jnp.int32), _E)
    active = t_grid * _TM < sizes[e_grid]
    cand = jnp.argsort(~active, stable=True)
    step_e = e_grid[cand][:max_steps]
    step_t = t_grid[cand][:max_steps]
    nact = jnp.sum(active.astype(jnp.int32))
    sidx = jnp.arange(max_steps, dtype=jnp.int32)
    last = nact - 1
    step_e_enc = jnp.where(
        sidx < nact, step_e + 1, -(step_e[last] + 1)
    ).astype(jnp.int32)
    step_t = jnp.where(sidx < nact, step_t, step_t[last])
    step_b = (al_start[jnp.abs(step_e_enc) - 1] // _TM).astype(jnp.int32) + step_t
    step_bo = jnp.where(sidx < nact, step_b, s_pad // _TM).astype(jnp.int32)

    x_sorted = _sc_gather(x_flat, gidx)

    out_sorted = _run_ffn(
        x_sorted, W_gate, W_up, W_down, step_e_enc, step_b, step_bo, max_steps
    )

    ab_rows = _sc_gather(out_sorted, ab_idx)
    out = _run_combine(
        x_flat, W_shared_gate, W_shared_up, W_shared_down, ab_rows, sc_out
    )
    return out.reshape(bx, tx, d)


# speedup vs baseline: 1.0724x; 1.0724x over previous
import functools

import jax
import jax.numpy as jnp
from jax import lax
from jax.experimental import pallas as pl
from jax.experimental.pallas import tpu as pltpu
from jax.experimental.pallas import tpu_sc as plsc

_E = 8
_TOPK = 2
_TM = 256
_LANES = 128


def _router_body(x_ref, wg_ref, idx_ref, sc_ref):
    x = x_ref[...]
    wg = wg_ref[...]
    logits = jnp.dot(x, wg, preferred_element_type=jnp.float32)
    col = lax.broadcasted_iota(jnp.int32, logits.shape, 1)
    valid = col < _E
    logits = jnp.where(valid, logits, jnp.float32(-1e30))
    m = jnp.max(logits, axis=1, keepdims=True)
    p = jnp.exp(logits - m)
    p = jnp.where(valid, p, 0.0)
    p = p / jnp.sum(p, axis=1, keepdims=True)
    s1 = jnp.max(p, axis=1, keepdims=True)
    i1 = jnp.min(jnp.where(p == s1, col, _LANES), axis=1, keepdims=True)
    p2 = jnp.where(col == i1, jnp.float32(-1.0), p)
    s2 = jnp.max(p2, axis=1, keepdims=True)
    i2 = jnp.min(jnp.where(p2 == s2, col, _LANES), axis=1, keepdims=True)
    idx_ref[...] = jnp.where(col == 0, i1, jnp.where(col == 1, i2, 0))
    sc_ref[...] = jnp.where(col == 0, s1, jnp.where(col == 1, s2, 0.0))


def _run_router(x_flat, W_g):
    n, d = x_flat.shape
    wg_pad = jnp.zeros((d, _LANES), jnp.float32).at[:, :_E].set(W_g)
    grid = (n // _TM,)
    return pl.pallas_call(
        _router_body,
        grid=grid,
        in_specs=[
            pl.BlockSpec((_TM, d), lambda i: (i, 0)),
            pl.BlockSpec((d, _LANES), lambda i: (0, 0)),
        ],
        out_specs=[
            pl.BlockSpec((_TM, _LANES), lambda i: (i, 0)),
            pl.BlockSpec((_TM, _LANES), lambda i: (i, 0)),
        ],
        out_shape=[
            jax.ShapeDtypeStruct((n, _LANES), jnp.int32),
            jax.ShapeDtypeStruct((n, _LANES), jnp.float32),
        ],
    )(x_flat, wg_pad)


def _sc_gather(table, idx):
    n_rows = idx.shape[0]
    d = table.shape[1]
    info = plsc.get_sparse_core_info()
    nw = info.num_cores * info.num_subcores
    per_w = n_rows // nw
    ch = next(c for c in (24, 16, 8) if per_w % c == 0)
    n_ch = per_w // ch
    mesh = plsc.VectorSubcoreMesh(core_axis_name="c", subcore_axis_name="s")

    @functools.partial(
        pl.kernel,
        mesh=mesh,
        out_type=jax.ShapeDtypeStruct((n_rows, d), jnp.float32),
        scratch_types=[
            pltpu.VMEM((per_w,), jnp.int32),
            pltpu.VMEM((2, ch, d), jnp.float32),
            pltpu.SemaphoreType.DMA,
            pltpu.SemaphoreType.DMA,
            pltpu.SemaphoreType.DMA,
        ],
    )
    def k(table_hbm, idx_hbm, out_hbm, idx_v, rows_v, gsem0, gsem1, osem):
        wid = lax.axis_index("s") * info.num_cores + lax.axis_index("c")
        base = wid * per_w
        pltpu.sync_copy(idx_hbm.at[pl.ds(base, per_w)], idx_v)
        gsems = [gsem0, gsem1]

        def gath(c):
            return pltpu.async_copy(
                table_hbm.at[idx_v.at[pl.ds(c * ch, ch)]],
                rows_v.at[c % 2],
                gsems[c % 2],
            )

        gath(0)
        outs = []
        for c in range(n_ch):
            pltpu.make_async_copy(
                table_hbm.at[idx_v.at[pl.ds(c * ch, ch)]],
                rows_v.at[c % 2],
                gsems[c % 2],
            ).wait()
            if c + 1 < n_ch:
                if c - 1 >= 0:
                    outs[c - 1].wait()
                gath(c + 1)
            outs.append(
                pltpu.async_copy(
                    rows_v.at[c % 2],
                    out_hbm.at[pl.ds(base + c * ch, ch)],
                    osem,
                )
            )
        outs[n_ch - 2].wait()
        outs[n_ch - 1].wait()

    return k(table, idx)


def _ffn_body(se_ref, sb_ref, sbo_ref, x_ref, wg_ref, wu_ref, wd_ref, out_ref):
    del sb_ref, sbo_ref
    s = pl.program_id(0)

    @pl.when(se_ref[s] > 0)
    def _work():
        x = x_ref[...]
        g = jnp.dot(x, wg_ref[0], preferred_element_type=jnp.float32)
        u = jnp.dot(x, wu_ref[0], preferred_element_type=jnp.float32)
        h = (g * jax.nn.sigmoid(g)) * u
        out_ref[...] = jnp.dot(h, wd_ref[0], preferred_element_type=jnp.float32)


def _run_ffn(x_sorted, W_gate, W_up, W_down, step_e, step_b, step_bo, max_steps):
    s_pad, d = x_sorted.shape
    f = W_gate.shape[-1]
    grid_spec = pltpu.PrefetchScalarGridSpec(
        num_scalar_prefetch=3,
        grid=(max_steps,),
        in_specs=[
            pl.BlockSpec((_TM, d), lambda s, se, sb, sbo: (sb[s], 0)),
            pl.BlockSpec((1, d, f), lambda s, se, sb, sbo: (jnp.abs(se[s]) - 1, 0, 0)),
            pl.BlockSpec((1, d, f), lambda s, se, sb, sbo: (jnp.abs(se[s]) - 1, 0, 0)),
            pl.BlockSpec((1, f, d), lambda s, se, sb, sbo: (jnp.abs(se[s]) - 1, 0, 0)),
        ],
        out_specs=pl.BlockSpec((_TM, d), lambda s, se, sb, sbo: (sbo[s], 0)),
    )
    return pl.pallas_call(
        _ffn_body,
        grid_spec=grid_spec,
        out_shape=jax.ShapeDtypeStruct((s_pad + _TM, d), jnp.float32),
    )(step_e, step_b, step_bo, x_sorted, W_gate, W_up, W_down)


def _comb_body(x_ref, wsg_ref, wsu_ref, wsd_ref, a_ref, b_ref, sc_ref, out_ref):
    x = x_ref[...]
    g = jnp.dot(x, wsg_ref[...], preferred_element_type=jnp.float32)
    u = jnp.dot(x, wsu_ref[...], preferred_element_type=jnp.float32)
    h = (g * jax.nn.sigmoid(g)) * u
    out = jnp.dot(h, wsd_ref[...], preferred_element_type=jnp.float32)
    out_ref[...] = out + a_ref[...] * sc_ref[:, 0:1] + b_ref[...] * sc_ref[:, 1:2]


def _run_combine(x_flat, Wsg, Wsu, Wsd, ab_rows, sc_out):
    n, d = x_flat.shape
    f = Wsg.shape[-1]
    n_t = n // _TM
    return pl.pallas_call(
        _comb_body,
        grid=(n_t,),
        in_specs=[
            pl.BlockSpec((_TM, d), lambda i: (i, 0)),
            pl.BlockSpec((d, f), lambda i: (0, 0)),
            pl.BlockSpec((d, f), lambda i: (0, 0)),
            pl.BlockSpec((f, d), lambda i: (0, 0)),
            pl.BlockSpec((_TM, d), lambda i: (i, 0)),
            pl.BlockSpec((_TM, d), lambda i, n_t=n_t: (i + n_t, 0)),
            pl.BlockSpec((_TM, _LANES), lambda i: (i, 0)),
        ],
        out_specs=pl.BlockSpec((_TM, d), lambda i: (i, 0)),
        out_shape=jax.ShapeDtypeStruct((n, d), jnp.float32),
    )(x_flat, Wsg, Wsu, Wsd, ab_rows, ab_rows, sc_out)


def kernel(x, W_g, W_gate, W_up, W_down, W_shared_gate, W_shared_up, W_shared_down):
    bx, tx, d = x.shape
    n = bx * tx
    x_flat = x.reshape(n, d)
    s = n * _TOPK
    s_pad = s + _E * _TM
    max_steps = s_pad // _TM
    maxt = n // _TM

    idx_out, sc_out = _run_router(x_flat, W_g)

    flat_e = idx_out[:, :_TOPK].reshape(-1)
    flat_t = (jnp.arange(s, dtype=jnp.int32) // _TOPK).astype(jnp.int32)
    order = jnp.argsort(flat_e, stable=True)
    sorted_e = flat_e[order]
    sizes = jnp.bincount(flat_e, length=_E)
    start = jnp.concatenate([jnp.zeros((1,), sizes.dtype), jnp.cumsum(sizes)[:-1]])
    al_sizes = ((sizes + _TM - 1) // _TM) * _TM
    al_start = jnp.concatenate(
        [jnp.zeros((1,), al_sizes.dtype), jnp.cumsum(al_sizes)[:-1]]
    )
    rank = jnp.arange(s, dtype=sizes.dtype) - start[sorted_e]
    dest = (al_start[sorted_e] + rank).astype(jnp.int32)
    inv = jnp.zeros((s,), jnp.int32).at[order].set(dest)
    ab_idx = jnp.concatenate([inv[0::2], inv[1::2]])
    pad_idx = (jnp.arange(s_pad, dtype=jnp.int32) * 7) % n
    gidx = pad_idx.at[dest].set(flat_t[order])

    e_grid = jnp.repeat(jnp.arange(_E, dtype=jnp.int32), maxt)
    t_grid = jnp.tile(jnp.arange(maxt, dtype=jnp.int32), _E)
    active = t_grid * _TM < sizes[e_grid]
    cand = jnp.argsort(~active, stable=True)
    step_e = e_grid[cand][:max_steps]
    step_t = t_grid[cand][:max_steps]
    nact = jnp.sum(active.astype(jnp.int32))
    sidx = jnp.arange(max_steps, dtype=jnp.int32)
    last = nact - 1
    step_e_enc = jnp.where(
        sidx < nact, step_e + 1, -(step_e[last] + 1)
    ).astype(jnp.int32)
    step_t = jnp.where(sidx < nact, step_t, step_t[last])
    step_b = (al_start[jnp.abs(step_e_enc) - 1] // _TM).astype(jnp.int32) + step_t
    step_bo = jnp.where(sidx < nact, step_b, s_pad // _TM).astype(jnp.int32)

    x_sorted = _sc_gather(x_flat, gidx)

    out_sorted = _run_ffn(
        x_sorted, W_gate, W_up, W_down, step_e_enc, step_b, step_bo, max_steps
    )

    ab_rows = _sc_gather(out_sorted, ab_idx)
    out = _run_combine(
        x_flat, W_shared_gate, W_shared_up, W_shared_down, ab_rows, sc_out
    )
    return out.reshape(bx, tx, d)


# speedup vs baseline: 1.0801x; 1.0072x over previous
import functools

import jax
import jax.numpy as jnp
from jax import lax
from jax.experimental import pallas as pl
from jax.experimental.pallas import tpu as pltpu
from jax.experimental.pallas import tpu_sc as plsc

_E = 8
_TOPK = 2
_TM = 256
_LANES = 128


def _router_body(x_ref, wg_ref, idx_ref, sc_ref):
    x = x_ref[...]
    wg = wg_ref[...]
    logits = jnp.dot(x, wg, preferred_element_type=jnp.float32)
    col = lax.broadcasted_iota(jnp.int32, logits.shape, 1)
    valid = col < _E
    logits = jnp.where(valid, logits, jnp.float32(-1e30))
    m = jnp.max(logits, axis=1, keepdims=True)
    p = jnp.exp(logits - m)
    p = jnp.where(valid, p, 0.0)
    p = p / jnp.sum(p, axis=1, keepdims=True)
    s1 = jnp.max(p, axis=1, keepdims=True)
    i1 = jnp.min(jnp.where(p == s1, col, _LANES), axis=1, keepdims=True)
    p2 = jnp.where(col == i1, jnp.float32(-1.0), p)
    s2 = jnp.max(p2, axis=1, keepdims=True)
    i2 = jnp.min(jnp.where(p2 == s2, col, _LANES), axis=1, keepdims=True)
    idx_ref[...] = jnp.where(col == 0, i1, jnp.where(col == 1, i2, 0))
    sc_ref[...] = jnp.where(col == 0, s1, jnp.where(col == 1, s2, 0.0))


def _run_router(x_flat, W_g):
    n, d = x_flat.shape
    wg_pad = jnp.zeros((d, _LANES), jnp.float32).at[:, :_E].set(W_g)
    grid = (n // _TM,)
    return pl.pallas_call(
        _router_body,
        grid=grid,
        in_specs=[
            pl.BlockSpec((_TM, d), lambda i: (i, 0)),
            pl.BlockSpec((d, _LANES), lambda i: (0, 0)),
        ],
        out_specs=[
            pl.BlockSpec((_TM, _LANES), lambda i: (i, 0)),
            pl.BlockSpec((_TM, _LANES), lambda i: (i, 0)),
        ],
        out_shape=[
            jax.ShapeDtypeStruct((n, _LANES), jnp.int32),
            jax.ShapeDtypeStruct((n, _LANES), jnp.float32),
        ],
    )(x_flat, wg_pad)


def _sc_gather(table, idx):
    n_rows = idx.shape[0]
    d = table.shape[1]
    info = plsc.get_sparse_core_info()
    nw = info.num_cores * info.num_subcores
    per_w = n_rows // nw
    ch = next(c for c in (24, 16, 8) if per_w % c == 0)
    n_ch = per_w // ch
    mesh = plsc.VectorSubcoreMesh(core_axis_name="c", subcore_axis_name="s")

    @functools.partial(
        pl.kernel,
        mesh=mesh,
        out_type=jax.ShapeDtypeStruct((n_rows, d), jnp.float32),
        scratch_types=[
            pltpu.VMEM((per_w,), jnp.int32),
            pltpu.VMEM((2, ch, d), jnp.float32),
            pltpu.SemaphoreType.DMA,
            pltpu.SemaphoreType.DMA,
            pltpu.SemaphoreType.DMA,
        ],
    )
    def k(table_hbm, idx_hbm, out_hbm, idx_v, rows_v, gsem0, gsem1, osem):
        wid = lax.axis_index("s") * info.num_cores + lax.axis_index("c")
        base = wid * per_w
        pltpu.sync_copy(idx_hbm.at[pl.ds(base, per_w)], idx_v)
        gsems = [gsem0, gsem1]

        def gath(c):
            return pltpu.async_copy(
                table_hbm.at[idx_v.at[pl.ds(c * ch, ch)]],
                rows_v.at[c % 2],
                gsems[c % 2],
            )

        gath(0)
        outs = []
        for c in range(n_ch):
            pltpu.make_async_copy(
                table_hbm.at[idx_v.at[pl.ds(c * ch, ch)]],
                rows_v.at[c % 2],
                gsems[c % 2],
            ).wait()
            if c + 1 < n_ch:
                if c - 1 >= 0:
                    outs[c - 1].wait()
                gath(c + 1)
            outs.append(
                pltpu.async_copy(
                    rows_v.at[c % 2],
                    out_hbm.at[pl.ds(base + c * ch, ch)],
                    osem,
                )
            )
        outs[n_ch - 2].wait()
        outs[n_ch - 1].wait()

    return k(table, idx)


def _ffn_body(se_ref, sb_ref, sbo_ref, x_ref, wg_ref, wu_ref, wd_ref, out_ref):
    del sb_ref, sbo_ref
    s = pl.program_id(0)

    @pl.when(se_ref[s] > 0)
    def _work():
        x = x_ref[...]
        g = jnp.dot(x, wg_ref[0], preferred_element_type=jnp.float32)
        u = jnp.dot(x, wu_ref[0], preferred_element_type=jnp.float32)
        h = (g * jax.nn.sigmoid(g)) * u
        out_ref[...] = jnp.dot(h, wd_ref[0], preferred_element_type=jnp.float32)


def _run_ffn(x_sorted, W_gate, W_up, W_down, step_e, step_b, step_bo, max_steps):
    s_pad, d = x_sorted.shape
    f = W_gate.shape[-1]
    grid_spec = pltpu.PrefetchScalarGridSpec(
        num_scalar_prefetch=3,
        grid=(max_steps,),
        in_specs=[
            pl.BlockSpec((_TM, d), lambda s, se, sb, sbo: (sb[s], 0)),
            pl.BlockSpec((1, d, f), lambda s, se, sb, sbo: (jnp.abs(se[s]) - 1, 0, 0)),
            pl.BlockSpec((1, d, f), lambda s, se, sb, sbo: (jnp.abs(se[s]) - 1, 0, 0)),
            pl.BlockSpec((1, f, d), lambda s, se, sb, sbo: (jnp.abs(se[s]) - 1, 0, 0)),
        ],
        out_specs=pl.BlockSpec((_TM, d), lambda s, se, sb, sbo: (sbo[s], 0)),
    )
    return pl.pallas_call(
        _ffn_body,
        grid_spec=grid_spec,
        out_shape=jax.ShapeDtypeStruct((s_pad + _TM, d), jnp.float32),
    )(step_e, step_b, step_bo, x_sorted, W_gate, W_up, W_down)


def _shared_body(x_ref, wsg_ref, wsu_ref, wsd_ref, out_ref):
    x = x_ref[...]
    g = jnp.dot(x, wsg_ref[...], preferred_element_type=jnp.float32)
    u = jnp.dot(x, wsu_ref[...], preferred_element_type=jnp.float32)
    h = (g * jax.nn.sigmoid(g)) * u
    out_ref[...] = jnp.dot(h, wsd_ref[...], preferred_element_type=jnp.float32)


def _run_shared(x_flat, Wsg, Wsu, Wsd):
    n, d = x_flat.shape
    f = Wsg.shape[-1]
    return pl.pallas_call(
        _shared_body,
        grid=(n // _TM,),
        in_specs=[
            pl.BlockSpec((_TM, d), lambda i: (i, 0)),
            pl.BlockSpec((d, f), lambda i: (0, 0)),
            pl.BlockSpec((d, f), lambda i: (0, 0)),
            pl.BlockSpec((f, d), lambda i: (0, 0)),
        ],
        out_specs=pl.BlockSpec((_TM, d), lambda i: (i, 0)),
        out_shape=jax.ShapeDtypeStruct((n, d), jnp.float32),
    )(x_flat, Wsg, Wsu, Wsd)


def _comb_body(sh_ref, a_ref, b_ref, sc_ref, out_ref):
    out_ref[...] = (
        sh_ref[...]
        + a_ref[...] * sc_ref[:, 0:1]
        + b_ref[...] * sc_ref[:, 1:2]
    )


def _run_combine(shared, ab_rows, sc_out):
    n, d = shared.shape
    n_t = n // _TM
    return pl.pallas_call(
        _comb_body,
        grid=(n_t,),
        in_specs=[
            pl.BlockSpec((_TM, d), lambda i: (i, 0)),
            pl.BlockSpec((_TM, d), lambda i: (i, 0)),
            pl.BlockSpec((_TM, d), lambda i, n_t=n_t: (i + n_t, 0)),
            pl.BlockSpec((_TM, _LANES), lambda i: (i, 0)),
        ],
        out_specs=pl.BlockSpec((_TM, d), lambda i: (i, 0)),
        out_shape=jax.ShapeDtypeStruct((n, d), jnp.float32),
    )(shared, ab_rows, ab_rows, sc_out)


def kernel(x, W_g, W_gate, W_up, W_down, W_shared_gate, W_shared_up, W_shared_down):
    bx, tx, d = x.shape
    n = bx * tx
    x_flat = x.reshape(n, d)
    s = n * _TOPK
    s_pad = s + _E * _TM
    max_steps = s_pad // _TM
    maxt = n // _TM

    idx_out, sc_out = _run_router(x_flat, W_g)

    flat_e = idx_out[:, :_TOPK].reshape(-1)
    flat_t = (jnp.arange(s, dtype=jnp.int32) // _TOPK).astype(jnp.int32)
    order = jnp.argsort(flat_e, stable=True)
    sorted_e = flat_e[order]
    sizes = jnp.bincount(flat_e, length=_E)
    start = jnp.concatenate([jnp.zeros((1,), sizes.dtype), jnp.cumsum(sizes)[:-1]])
    al_sizes = ((sizes + _TM - 1) // _TM) * _TM
    al_start = jnp.concatenate(
        [jnp.zeros((1,), al_sizes.dtype), jnp.cumsum(al_sizes)[:-1]]
    )
    rank = jnp.arange(s, dtype=sizes.dtype) - start[sorted_e]
    dest = (al_start[sorted_e] + rank).astype(jnp.int32)
    inv = jnp.zeros((s,), jnp.int32).at[order].set(dest)
    ab_idx = jnp.concatenate([inv[0::2], inv[1::2]])
    pad_idx = (jnp.arange(s_pad, dtype=jnp.int32) * 7) % n
    gidx = pad_idx.at[dest].set(flat_t[order])

    e_grid = jnp.repeat(jnp.arange(_E, dtype=jnp.int32), maxt)
    t_grid = jnp.tile(jnp.arange(maxt, dtype=jnp.int32), _E)
    active = t_grid * _TM < sizes[e_grid]
    cand = jnp.argsort(~active, stable=True)
    step_e = e_grid[cand][:max_steps]
    step_t = t_grid[cand][:max_steps]
    nact = jnp.sum(active.astype(jnp.int32))
    sidx = jnp.arange(max_steps, dtype=jnp.int32)
    last = nact - 1
    step_e_enc = jnp.where(
        sidx < nact, step_e + 1, -(step_e[last] + 1)
    ).astype(jnp.int32)
    step_t = jnp.where(sidx < nact, step_t, step_t[last])
    step_b = (al_start[jnp.abs(step_e_enc) - 1] // _TM).astype(jnp.int32) + step_t
    step_bo = jnp.where(sidx < nact, step_b, s_pad // _TM).astype(jnp.int32)

    x_sorted = _sc_gather(x_flat, gidx)

    out_sorted = _run_ffn(
        x_sorted, W_gate, W_up, W_down, step_e_enc, step_b, step_bo, max_steps
    )

    ab_rows = _sc_gather(out_sorted, ab_idx)
    shared = _run_shared(x_flat, W_shared_gate, W_shared_up, W_shared_down)
    out = _run_combine(shared, ab_rows, sc_out)
    return out.reshape(bx, tx, d)


# speedup vs baseline: 1.0923x; 1.0113x over previous
import functools

import jax
import jax.numpy as jnp
from jax import lax
from jax.experimental import pallas as pl
from jax.experimental.pallas import tpu as pltpu
from jax.experimental.pallas import tpu_sc as plsc

_E = 8
_TOPK = 2
_TM = 256
_LANES = 128


def _router_body(x_ref, wg_ref, idx_ref, sc_ref):
    x = x_ref[...]
    wg = wg_ref[...]
    logits = jnp.dot(x, wg, preferred_element_type=jnp.float32)
    col = lax.broadcasted_iota(jnp.int32, logits.shape, 1)
    valid = col < _E
    logits = jnp.where(valid, logits, jnp.float32(-1e30))
    m = jnp.max(logits, axis=1, keepdims=True)
    p = jnp.exp(logits - m)
    p = jnp.where(valid, p, 0.0)
    p = p / jnp.sum(p, axis=1, keepdims=True)
    s1 = jnp.max(p, axis=1, keepdims=True)
    i1 = jnp.min(jnp.where(p == s1, col, _LANES), axis=1, keepdims=True)
    p2 = jnp.where(col == i1, jnp.float32(-1.0), p)
    s2 = jnp.max(p2, axis=1, keepdims=True)
    i2 = jnp.min(jnp.where(p2 == s2, col, _LANES), axis=1, keepdims=True)
    idx_ref[...] = jnp.where(col == 0, i1, jnp.where(col == 1, i2, 0))
    sc_ref[...] = jnp.where(col == 0, s1, jnp.where(col == 1, s2, 0.0))


def _run_router(x_flat, W_g):
    n, d = x_flat.shape
    tm = 512
    wg_pad = jnp.zeros((d, _LANES), jnp.float32).at[:, :_E].set(W_g)
    grid = (n // tm,)
    return pl.pallas_call(
        _router_body,
        grid=grid,
        in_specs=[
            pl.BlockSpec((tm, d), lambda i: (i, 0)),
            pl.BlockSpec((d, _LANES), lambda i: (0, 0)),
        ],
        out_specs=[
            pl.BlockSpec((tm, _LANES), lambda i: (i, 0)),
            pl.BlockSpec((tm, _LANES), lambda i: (i, 0)),
        ],
        out_shape=[
            jax.ShapeDtypeStruct((n, _LANES), jnp.int32),
            jax.ShapeDtypeStruct((n, _LANES), jnp.float32),
        ],
    )(x_flat, wg_pad)


def _sc_gather(table, idx):
    n_rows = idx.shape[0]
    d = table.shape[1]
    info = plsc.get_sparse_core_info()
    nw = info.num_cores * info.num_subcores
    per_w = n_rows // nw
    ch = next(c for c in (24, 16, 8) if per_w % c == 0)
    n_ch = per_w // ch
    mesh = plsc.VectorSubcoreMesh(core_axis_name="c", subcore_axis_name="s")

    @functools.partial(
        pl.kernel,
        mesh=mesh,
        out_type=jax.ShapeDtypeStruct((n_rows, d), jnp.float32),
        scratch_types=[
            pltpu.VMEM((per_w,), jnp.int32),
            pltpu.VMEM((2, ch, d), jnp.float32),
            pltpu.SemaphoreType.DMA,
            pltpu.SemaphoreType.DMA,
            pltpu.SemaphoreType.DMA,
        ],
    )
    def k(table_hbm, idx_hbm, out_hbm, idx_v, rows_v, gsem0, gsem1, osem):
        wid = lax.axis_index("s") * info.num_cores + lax.axis_index("c")
        base = wid * per_w
        pltpu.sync_copy(idx_hbm.at[pl.ds(base, per_w)], idx_v)
        gsems = [gsem0, gsem1]

        def gath(c):
            return pltpu.async_copy(
                table_hbm.at[idx_v.at[pl.ds(c * ch, ch)]],
                rows_v.at[c % 2],
                gsems[c % 2],
            )

        gath(0)
        outs = []
        for c in range(n_ch):
            pltpu.make_async_copy(
                table_hbm.at[idx_v.at[pl.ds(c * ch, ch)]],
                rows_v.at[c % 2],
                gsems[c % 2],
            ).wait()
            if c + 1 < n_ch:
                if c - 1 >= 0:
                    outs[c - 1].wait()
                gath(c + 1)
            outs.append(
                pltpu.async_copy(
                    rows_v.at[c % 2],
                    out_hbm.at[pl.ds(base + c * ch, ch)],
                    osem,
                )
            )
        outs[n_ch - 2].wait()
        outs[n_ch - 1].wait()

    return k(table, idx)


def _ffn_body(se_ref, sb_ref, sbo_ref, x_ref, wg_ref, wu_ref, wd_ref, out_ref):
    del sb_ref, sbo_ref
    s = pl.program_id(0)

    @pl.when(se_ref[s] > 0)
    def _work():
        x = x_ref[...]
        g = jnp.dot(x, wg_ref[0], preferred_element_type=jnp.float32)
        u = jnp.dot(x, wu_ref[0], preferred_element_type=jnp.float32)
        h = (g * jax.nn.sigmoid(g)) * u
        out_ref[...] = jnp.dot(h, wd_ref[0], preferred_element_type=jnp.float32)


def _run_ffn(x_sorted, W_gate, W_up, W_down, step_e, step_b, step_bo, max_steps):
    s_pad, d = x_sorted.shape
    f = W_gate.shape[-1]
    grid_spec = pltpu.PrefetchScalarGridSpec(
        num_scalar_prefetch=3,
        grid=(max_steps,),
        in_specs=[
            pl.BlockSpec((_TM, d), lambda s, se, sb, sbo: (sb[s], 0)),
            pl.BlockSpec((1, d, f), lambda s, se, sb, sbo: (jnp.abs(se[s]) - 1, 0, 0)),
            pl.BlockSpec((1, d, f), lambda s, se, sb, sbo: (jnp.abs(se[s]) - 1, 0, 0)),
            pl.BlockSpec((1, f, d), lambda s, se, sb, sbo: (jnp.abs(se[s]) - 1, 0, 0)),
        ],
        out_specs=pl.BlockSpec((_TM, d), lambda s, se, sb, sbo: (sbo[s], 0)),
    )
    return pl.pallas_call(
        _ffn_body,
        grid_spec=grid_spec,
        out_shape=jax.ShapeDtypeStruct((s_pad + _TM, d), jnp.float32),
    )(step_e, step_b, step_bo, x_sorted, W_gate, W_up, W_down)


def _shared_body(x_ref, wsg_ref, wsu_ref, wsd_ref, out_ref):
    x = x_ref[...]
    g = jnp.dot(x, wsg_ref[...], preferred_element_type=jnp.float32)
    u = jnp.dot(x, wsu_ref[...], preferred_element_type=jnp.float32)
    h = (g * jax.nn.sigmoid(g)) * u
    out_ref[...] = jnp.dot(h, wsd_ref[...], preferred_element_type=jnp.float32)


def _run_shared(x_flat, Wsg, Wsu, Wsd):
    n, d = x_flat.shape
    f = Wsg.shape[-1]
    return pl.pallas_call(
        _shared_body,
        grid=(n // _TM,),
        in_specs=[
            pl.BlockSpec((_TM, d), lambda i: (i, 0)),
            pl.BlockSpec((d, f), lambda i: (0, 0)),
            pl.BlockSpec((d, f), lambda i: (0, 0)),
            pl.BlockSpec((f, d), lambda i: (0, 0)),
        ],
        out_specs=pl.BlockSpec((_TM, d), lambda i: (i, 0)),
        out_shape=jax.ShapeDtypeStruct((n, d), jnp.float32),
    )(x_flat, Wsg, Wsu, Wsd)


def _comb_body(sh_ref, a_ref, b_ref, sc_ref, out_ref):
    out_ref[...] = (
        sh_ref[...]
        + a_ref[...] * sc_ref[:, 0:1]
        + b_ref[...] * sc_ref[:, 1:2]
    )


def _run_combine(shared, ab_rows, sc_out):
    n, d = shared.shape
    n_t = n // _TM
    return pl.pallas_call(
        _comb_body,
        grid=(n_t,),
        in_specs=[
            pl.BlockSpec((_TM, d), lambda i: (i, 0)),
            pl.BlockSpec((_TM, d), lambda i: (i, 0)),
            pl.BlockSpec((_TM, d), lambda i, n_t=n_t: (i + n_t, 0)),
            pl.BlockSpec((_TM, _LANES), lambda i: (i, 0)),
        ],
        out_specs=pl.BlockSpec((_TM, d), lambda i: (i, 0)),
        out_shape=jax.ShapeDtypeStruct((n, d), jnp.float32),
    )(shared, ab_rows, ab_rows, sc_out)


def kernel(x, W_g, W_gate, W_up, W_down, W_shared_gate, W_shared_up, W_shared_down):
    bx, tx, d = x.shape
    n = bx * tx
    x_flat = x.reshape(n, d)
    s = n * _TOPK
    s_pad = s + _E * _TM
    max_steps = s_pad // _TM
    maxt = n // _TM

    idx_out, sc_out = _run_router(x_flat, W_g)

    flat_e = idx_out[:, :_TOPK].reshape(-1)
    flat_t = (jnp.arange(s, dtype=jnp.int32) // _TOPK).astype(jnp.int32)
    order = jnp.argsort(flat_e, stable=True)
    sorted_e = flat_e[order]
    sizes = jnp.bincount(flat_e, length=_E)
    start = jnp.concatenate([jnp.zeros((1,), sizes.dtype), jnp.cumsum(sizes)[:-1]])
    al_sizes = ((sizes + _TM - 1) // _TM) * _TM
    al_start = jnp.concatenate(
        [jnp.zeros((1,), al_sizes.dtype), jnp.cumsum(al_sizes)[:-1]]
    )
    rank = jnp.arange(s, dtype=sizes.dtype) - start[sorted_e]
    dest = (al_start[sorted_e] + rank).astype(jnp.int32)
    inv = jnp.zeros((s,), jnp.int32).at[order].set(dest)
    ab_idx = jnp.concatenate([inv[0::2], inv[1::2]])
    pad_idx = (jnp.arange(s_pad, dtype=jnp.int32) * 7) % n
    gidx = pad_idx.at[dest].set(flat_t[order])

    e_grid = jnp.repeat(jnp.arange(_E, dtype=jnp.int32), maxt)
    t_grid = jnp.tile(jnp.arange(maxt, dtype=jnp.int32), _E)
    active = t_grid * _TM < sizes[e_grid]
    cand = jnp.argsort(~active, stable=True)
    step_e = e_grid[cand][:max_steps]
    step_t = t_grid[cand][:max_steps]
    nact = jnp.sum(active.astype(jnp.int32))
    sidx = jnp.arange(max_steps, dtype=jnp.int32)
    last = nact - 1
    step_e_enc = jnp.where(
        sidx < nact, step_e + 1, -(step_e[last] + 1)
    ).astype(jnp.int32)
    step_t = jnp.where(sidx < nact, step_t, step_t[last])
    step_b = (al_start[jnp.abs(step_e_enc) - 1] // _TM).astype(jnp.int32) + step_t
    step_bo = jnp.where(sidx < nact, step_b, s_pad // _TM).astype(jnp.int32)

    x_sorted = _sc_gather(x_flat, gidx)

    out_sorted = _run_ffn(
        x_sorted, W_gate, W_up, W_down, step_e_enc, step_b, step_bo, max_steps
    )

    ab_rows = _sc_gather(out_sorted, ab_idx)
    shared = _run_shared(x_flat, W_shared_gate, W_shared_up, W_shared_down)
    out = _run_combine(shared, ab_rows, sc_out)
    return out.reshape(bx, tx, d)
